# baseline (speedup 1.0000x reference)
import functools

import jax
import jax.numpy as jnp
from jax import lax
from jax.experimental import pallas as pl
from jax.experimental.pallas import tpu as pltpu
from jax.experimental.pallas import tpu_sc as plsc

LANES = 16
NUM_CORES = 2
NUM_SUBCORES = 16
NW = NUM_CORES * NUM_SUBCORES
IDX_CHUNK = 128


def _make_enmf_sc(n_per_w, n_chunks, d):
    mesh = plsc.VectorSubcoreMesh(core_axis_name="c", subcore_axis_name="s")

    @functools.partial(
        pl.kernel,
        mesh=mesh,
        compiler_params=pltpu.CompilerParams(
            needs_layout_passes=False, use_tc_tiling_on_sc=False),
        out_type=jax.ShapeDtypeStruct((NW, n_per_w), jnp.float32),
        scratch_types=[
            pltpu.VMEM((n_chunks, IDX_CHUNK), jnp.int32),
            pltpu.VMEM((n_chunks, IDX_CHUNK), jnp.int32),
            pltpu.VMEM((n_per_w, d), jnp.float32),
            pltpu.VMEM((n_per_w, d), jnp.float32),
            pltpu.VMEM((d,), jnp.float32),
            pltpu.VMEM((n_per_w,), jnp.float32),
            pltpu.SemaphoreType.DMA,
        ],
    )
    def k(users_hbm, items_hbm, ut_hbm, it_hbm, h_hbm, out_hbm,
          uidx_v, iidx_v, urows_v, irows_v, h_v, out_v, sem):
        wid = lax.axis_index("s") * NUM_CORES + lax.axis_index("c")
        pltpu.sync_copy(users_hbm.at[wid], uidx_v)
        pltpu.sync_copy(items_hbm.at[wid], iidx_v)
        pltpu.sync_copy(h_hbm, h_v)

        copies = []
        for j in range(n_chunks):
            dst = pl.ds(j * IDX_CHUNK, IDX_CHUNK)
            copies.append(pltpu.async_copy(
                ut_hbm.at[uidx_v.at[j]], urows_v.at[dst], sem))
            copies.append(pltpu.async_copy(
                it_hbm.at[iidx_v.at[j]], irows_v.at[dst], sem))
        for cp in copies:
            cp.wait()

        hv = h_v[...]
        hs = [hv[c] for c in range(d)]
        col_ids = [jnp.full((LANES,), c, jnp.int32) for c in range(d)]

        def body(g, carry):
            row_ids = g * LANES + lax.iota(jnp.int32, LANES)
            acc = jnp.zeros((LANES,), jnp.float32)
            for c in range(d):
                ucol = plsc.load_gather(urows_v, [row_ids, col_ids[c]])
                icol = plsc.load_gather(irows_v, [row_ids, col_ids[c]])
                acc = acc + ucol * icol * hs[c]
            out_v[pl.ds(g * LANES, LANES)] = acc
            return carry

        lax.fori_loop(0, n_per_w // LANES, body, 0)
        pltpu.sync_copy(out_v, out_hbm.at[wid])

    return k


def kernel(users, items, user_table, item_table, h):
    n = users.shape[0]
    d = user_table.shape[1]
    n_per_w = n // NW
    n_chunks = n_per_w // IDX_CHUNK
    users_r = users.reshape(NW, n_chunks, IDX_CHUNK)
    items_r = items.reshape(NW, n_chunks, IDX_CHUNK)
    out = _make_enmf_sc(n_per_w, n_chunks, d)(
        users_r, items_r, user_table, item_table, h)
    return out.reshape(n)
